# baseline (speedup 1.0000x reference)
import jax
import jax.numpy as jnp
from jax.experimental import pallas as pl
from jax.experimental.pallas import tpu as pltpu


def _mlp_stream_kernel(w1_ref, b1_ref, w2_ref, b2_ref, xa_ref, xh_ref,
                       o_ref, xb_buf, sem):
    i = pl.program_id(0)
    n = pl.num_programs(0)
    tile = xa_ref.shape[1]

    def rows89_copy(step, slot):
        return pltpu.make_async_copy(
            xh_ref.at[pl.ds(8, 2), pl.ds(step * tile, tile)],
            xb_buf.at[slot],
            sem.at[slot],
        )

    @pl.when(i == 0)
    def _():
        rows89_copy(0, 0).start()

    if n > 1:
        @pl.when(i == 0)
        def _():
            rows89_copy(1, 1).start()

    if n > 2:
        @pl.when(i == 0)
        def _():
            rows89_copy(2, 2).start()

    @pl.when(i + 3 < n)
    def _():
        rows89_copy(i + 3, (i + 3) % 4).start()

    rows89_copy(i, i % 4).wait()
    xb = xb_buf[i % 4]

    h = jax.lax.dot_general(
        w1_ref[:, 0:8], xa_ref[...], (((1,), (0,)), ((), ())),
        preferred_element_type=jnp.float32,
    ) + jax.lax.dot_general(
        w1_ref[:, 8:10], xb, (((1,), (0,)), ((), ())),
        preferred_element_type=jnp.float32,
    )
    b1c = jnp.transpose(b1_ref[...], (1, 0))
    h = jnp.maximum(h + b1c, 0.0)
    y = jax.lax.dot_general(
        w2_ref[...], h, (((1,), (0,)), ((), ())),
        preferred_element_type=jnp.float32,
    )
    o_ref[...] = y + b2_ref[...]


def _ceil_to(v, m):
    return ((v + m - 1) // m) * m


def kernel(x_t, w1, b1, w2, b2):
    F, B = x_t.shape
    assert F == 10, "expects 10 input features"

    tile = 262144
    b_pad = _ceil_to(B, 128)
    if b_pad <= tile:
        tile = b_pad
    else:
        n = -(-b_pad // tile)
        tile = _ceil_to(-(-b_pad // n), 128)
        b_pad = _ceil_to(b_pad, tile)

    x_t = x_t.astype(jnp.float32)
    if b_pad != B:
        x_t = jnp.pad(x_t, ((0, 0), (0, b_pad - B)))

    w1 = w1.astype(jnp.float32)
    b1r = b1.astype(jnp.float32).reshape(1, 5)
    w2r = w2.astype(jnp.float32).reshape(1, 5)
    b2r = b2.astype(jnp.float32).reshape(1, 1)

    const = lambda i: (0, 0)
    out = pl.pallas_call(
        _mlp_stream_kernel,
        out_shape=jax.ShapeDtypeStruct((1, b_pad), jnp.float32),
        grid=(b_pad // tile,),
        in_specs=[
            pl.BlockSpec((5, 10), const),
            pl.BlockSpec((1, 5), const),
            pl.BlockSpec((1, 5), const),
            pl.BlockSpec((1, 1), const),
            pl.BlockSpec((8, tile), lambda i: (0, i)),
            pl.BlockSpec(memory_space=pltpu.MemorySpace.HBM),
        ],
        out_specs=pl.BlockSpec((1, tile), lambda i: (0, i)),
        scratch_shapes=[
            pltpu.VMEM((4, 2, tile), jnp.float32),
            pltpu.SemaphoreType.DMA((4,)),
        ],
        compiler_params=pltpu.CompilerParams(
            dimension_semantics=("arbitrary",),
        ),
        cost_estimate=pl.CostEstimate(
            flops=120 * b_pad,
            transcendentals=0,
            bytes_accessed=44 * b_pad + 1024,
        ),
    )(w1, b1r, w2r, b2r, x_t, x_t)

    if b_pad == B:
        return out
    return out[:, :B]


# speedup vs baseline: 1.0351x; 1.0351x over previous
import jax
import jax.numpy as jnp
from jax.experimental import pallas as pl
from jax.experimental.pallas import tpu as pltpu


def _mlp_stream_kernel(w1_ref, b1_ref, w2_ref, b2_ref, xa_ref, xh_ref,
                       o_ref, xb_buf, sem, y_buf, wsem):
    i = pl.program_id(0)
    n = pl.num_programs(0)
    tile = xa_ref.shape[1]

    def rows89_copy(step, slot):
        return pltpu.make_async_copy(
            xh_ref.at[pl.ds(8, 2), pl.ds(step * tile, tile)],
            xb_buf.at[slot],
            sem.at[slot],
        )

    @pl.when(i == 0)
    def _():
        rows89_copy(0, 0).start()

    if n > 1:
        @pl.when(i == 0)
        def _():
            rows89_copy(1, 1).start()

    if n > 2:
        @pl.when(i == 0)
        def _():
            rows89_copy(2, 2).start()

    @pl.when(i + 3 < n)
    def _():
        rows89_copy(i + 3, (i + 3) % 4).start()

    rows89_copy(i, i % 4).wait()
    xb = xb_buf[i % 4]

    h = jax.lax.dot_general(
        w1_ref[:, 0:8], xa_ref[...], (((1,), (0,)), ((), ())),
        preferred_element_type=jnp.float32,
    ) + jax.lax.dot_general(
        w1_ref[:, 8:10], xb, (((1,), (0,)), ((), ())),
        preferred_element_type=jnp.float32,
    )
    b1c = jnp.transpose(b1_ref[...], (1, 0))
    h = jnp.maximum(h + b1c, 0.0)
    y = jax.lax.dot_general(
        w2_ref[...], h, (((1,), (0,)), ((), ())),
        preferred_element_type=jnp.float32,
    )

    def out_copy(step):
        return pltpu.make_async_copy(
            y_buf.at[step % 2],
            o_ref.at[0:1, pl.ds(step * tile, tile)],
            wsem.at[step % 2],
        )

    @pl.when(i >= 2)
    def _():
        out_copy(i - 2).wait()

    y_buf[i % 2] = y + b2_ref[...]
    out_copy(i).start()

    if n > 1:
        @pl.when(i == n - 1)
        def _():
            out_copy(i - 1).wait()
            out_copy(i).wait()
    else:
        @pl.when(i == 0)
        def _():
            out_copy(0).wait()


def _ceil_to(v, m):
    return ((v + m - 1) // m) * m


def kernel(x_t, w1, b1, w2, b2):
    F, B = x_t.shape
    assert F == 10, "expects 10 input features"

    tile = 262144
    b_pad = _ceil_to(B, 128)
    if b_pad <= tile:
        tile = b_pad
    else:
        n = -(-b_pad // tile)
        tile = _ceil_to(-(-b_pad // n), 128)
        b_pad = _ceil_to(b_pad, tile)

    x_t = x_t.astype(jnp.float32)
    if b_pad != B:
        x_t = jnp.pad(x_t, ((0, 0), (0, b_pad - B)))

    w1 = w1.astype(jnp.float32)
    b1r = b1.astype(jnp.float32).reshape(1, 5)
    w2r = w2.astype(jnp.float32).reshape(1, 5)
    b2r = b2.astype(jnp.float32).reshape(1, 1)

    const = lambda i: (0, 0)
    out = pl.pallas_call(
        _mlp_stream_kernel,
        out_shape=jax.ShapeDtypeStruct((1, b_pad), jnp.float32),
        grid=(b_pad // tile,),
        in_specs=[
            pl.BlockSpec((5, 10), const),
            pl.BlockSpec((1, 5), const),
            pl.BlockSpec((1, 5), const),
            pl.BlockSpec((1, 1), const),
            pl.BlockSpec((8, tile), lambda i: (0, i)),
            pl.BlockSpec(memory_space=pltpu.MemorySpace.HBM),
        ],
        out_specs=pl.BlockSpec(memory_space=pltpu.MemorySpace.HBM),
        scratch_shapes=[
            pltpu.VMEM((4, 2, tile), jnp.float32),
            pltpu.SemaphoreType.DMA((4,)),
            pltpu.VMEM((2, 1, tile), jnp.float32),
            pltpu.SemaphoreType.DMA((2,)),
        ],
        compiler_params=pltpu.CompilerParams(
            dimension_semantics=("arbitrary",),
        ),
        cost_estimate=pl.CostEstimate(
            flops=120 * b_pad,
            transcendentals=0,
            bytes_accessed=44 * b_pad + 1024,
        ),
    )(w1, b1r, w2r, b2r, x_t, x_t)

    if b_pad == B:
        return out
    return out[:, :B]
